# Initial kernel scaffold; baseline (speedup 1.0000x reference)
import jax
import jax.numpy as jnp
from jax.experimental import pallas as pl


def kernel(features, edge_index, edge_type, edge_matrix, W_ih, W_hh, b_ih, b_hh, W_out, b_out):
    raise NotImplementedError("write your pallas kernel here")


# speedup vs baseline: 25.4583x; 25.4583x over previous
import functools

import jax
import jax.numpy as jnp
from jax import lax
from jax.experimental import pallas as pl
from jax.experimental.pallas import tpu as pltpu
from jax.experimental.pallas import tpu_sc as plsc

N = 50000
E = 800000
MSG = 16
HID = 16
T = 16
C = 64

NC = 2
NS = 16
NW = NC * NS

CH = 128
K = 196
EPT = K * CH
E_PAD = NW * EPT
NB = 8
ACC = 50048
RPT = ACC // NS
RPT_FULL = RPT // CH
RPT_REM = RPT - RPT_FULL * CH


def _sc_scatter(g_tab, src_m, type_m, dst_m):
    mesh = plsc.VectorSubcoreMesh(core_axis_name="c", subcore_axis_name="s")

    @functools.partial(
        pl.kernel,
        out_type=jax.ShapeDtypeStruct((NC, ACC, MSG), jnp.float32),
        mesh=mesh,
        compiler_params=pltpu.CompilerParams(use_tc_tiling_on_sc=False),
        scratch_types=[
            pltpu.VMEM((K, CH), jnp.int32),
            pltpu.VMEM((K, CH), jnp.int32),
            pltpu.VMEM((NB, CH, MSG), jnp.float32),
            pltpu.VMEM_SHARED((ACC, MSG), jnp.float32),
        ] + [pltpu.SemaphoreType.DMA] * NB,
    )
    def body(g_hbm, src_hbm, type_hbm, dst_hbm, out_hbm,
             gidx_v, aux_v, rows_v, accum_sh, *sems):
        cid = lax.axis_index("c")
        sid = lax.axis_index("s")
        wid = sid * NC + cid

        zeros16 = jnp.zeros((MSG,), jnp.float32)

        def zrow(i, carry):
            rows_v[0, i, :] = zeros16
            return carry

        lax.fori_loop(0, CH, zrow, 0)
        z0 = sid * RPT

        def zblk(i, carry):
            pltpu.sync_copy(rows_v.at[0], accum_sh.at[pl.ds(z0 + i * CH, CH)])
            return carry

        lax.fori_loop(0, RPT_FULL, zblk, 0)
        pltpu.sync_copy(rows_v.at[0, pl.ds(0, RPT_REM)],
                        accum_sh.at[pl.ds(z0 + RPT_FULL * CH, RPT_REM)])
        plsc.subcore_barrier()

        pltpu.sync_copy(src_hbm.at[wid], gidx_v)
        pltpu.sync_copy(type_hbm.at[wid], aux_v)

        def gcomp(j, carry):
            for k in range(CH // 16):
                s = gidx_v[j, pl.ds(k * 16, 16)]
                t = aux_v[j, pl.ds(k * 16, 16)]
                gidx_v[j, pl.ds(k * 16, 16)] = s * T + t
            return carry

        lax.fori_loop(0, K, gcomp, 0)
        pltpu.sync_copy(dst_hbm.at[wid], aux_v)

        def fire(j, b):
            return pltpu.async_copy(g_hbm.at[gidx_v.at[j]], rows_v.at[b], sems[b])

        def scat(j, b):
            pltpu.sync_copy(rows_v.at[b], accum_sh.at[aux_v.at[j]], add=True)

        def group(gi, carry):
            j0 = gi * NB
            hs = [fire(j0 + b, b) for b in range(NB)]
            for b in range(NB):
                hs[b].wait()
                scat(j0 + b, b)
            return carry

        n_full = K // NB
        lax.fori_loop(0, n_full, group, 0)
        tail = K - n_full * NB
        if tail:
            hs = [fire(n_full * NB + b, b) for b in range(tail)]
            for b in range(tail):
                hs[b].wait()
                scat(n_full * NB + b, b)

        plsc.subcore_barrier()
        pltpu.sync_copy(accum_sh.at[pl.ds(z0, RPT)],
                        out_hbm.at[cid, pl.ds(z0, RPT)])

    return body(g_tab, src_m, type_m, dst_m)


def _tc_messages(features, B):
    BN = 2000
    grid = N // BN

    def body(f_ref, b_ref, g_ref):
        g_ref[...] = jnp.dot(f_ref[...], b_ref[...],
                             preferred_element_type=jnp.float32)

    return pl.pallas_call(
        body,
        grid=(grid,),
        in_specs=[
            pl.BlockSpec((BN, HID), lambda i: (i, 0)),
            pl.BlockSpec((HID, T * MSG), lambda i: (0, 0)),
        ],
        out_specs=pl.BlockSpec((BN, T * MSG), lambda i: (i, 0)),
        out_shape=jax.ShapeDtypeStruct((N, T * MSG), jnp.float32),
    )(features, B)


def _tc_gru_readout(m0, m1, features, wih_t, whh_t, bih, bhh, wout_t, bout):
    BN = 2000
    grid = N // BN

    def body(m0_ref, m1_ref, f_ref, wih_ref, whh_ref, bih_ref, bhh_ref,
             wout_ref, bout_ref, out_ref):
        m = m0_ref[...] + m1_ref[...]
        f = f_ref[...]
        gi = jnp.dot(m, wih_ref[...], preferred_element_type=jnp.float32) + bih_ref[...]
        gh = jnp.dot(f, whh_ref[...], preferred_element_type=jnp.float32) + bhh_ref[...]
        i_r, i_z, i_n = gi[:, 0:HID], gi[:, HID:2 * HID], gi[:, 2 * HID:3 * HID]
        h_r, h_z, h_n = gh[:, 0:HID], gh[:, HID:2 * HID], gh[:, 2 * HID:3 * HID]
        r = jax.nn.sigmoid(i_r + h_r)
        z = jax.nn.sigmoid(i_z + h_z)
        n = jnp.tanh(i_n + r * h_n)
        h = (1.0 - z) * n + z * f
        out_ref[...] = jnp.dot(h, wout_ref[...],
                               preferred_element_type=jnp.float32) + bout_ref[...]

    row = lambda i: (i, 0)
    fixed = lambda i: (0, 0)
    return pl.pallas_call(
        body,
        grid=(grid,),
        in_specs=[
            pl.BlockSpec((BN, MSG), row),
            pl.BlockSpec((BN, MSG), row),
            pl.BlockSpec((BN, HID), row),
            pl.BlockSpec((MSG, 3 * HID), fixed),
            pl.BlockSpec((HID, 3 * HID), fixed),
            pl.BlockSpec((1, 3 * HID), fixed),
            pl.BlockSpec((1, 3 * HID), fixed),
            pl.BlockSpec((HID, C), fixed),
            pl.BlockSpec((1, C), fixed),
        ],
        out_specs=pl.BlockSpec((BN, C), row),
        out_shape=jax.ShapeDtypeStruct((N, C), jnp.float32),
    )(m0, m1, features, wih_t, whh_t, bih, bhh, wout_t, bout)


def kernel(features, edge_index, edge_type, edge_matrix,
           W_ih, W_hh, b_ih, b_hh, W_out, b_out):
    src = edge_index[0]
    dst = edge_index[1]

    B = edge_matrix.reshape(T, MSG, HID).transpose(2, 0, 1).reshape(HID, T * MSG)
    g = _tc_messages(features, B)
    g_tab = g.reshape(N * T, MSG)

    pad = E_PAD - E
    src_m = jnp.concatenate([src, jnp.zeros((pad,), jnp.int32)]).reshape(NW, K, CH)
    type_m = jnp.concatenate([edge_type, jnp.zeros((pad,), jnp.int32)]).reshape(NW, K, CH)
    dst_m = jnp.concatenate([dst, jnp.full((pad,), N, jnp.int32)]).reshape(NW, K, CH)

    partials = _sc_scatter(g_tab, src_m, type_m, dst_m)
    m0 = partials[0, :N]
    m1 = partials[1, :N]

    return _tc_gru_readout(
        m0, m1, features,
        W_ih.T, W_hh.T, b_ih.reshape(1, -1), b_hh.reshape(1, -1),
        W_out.T, b_out.reshape(1, -1),
    )
